# baseline (speedup 1.0000x reference)
import functools

import jax
import jax.numpy as jnp
from jax import lax
from jax.experimental import pallas as pl
from jax.experimental.pallas import tpu as pltpu

_NEG = -1e30


def _routing_kernel(x_ref, gw_ref, bias_ref, rw_ref, *, n_exp_minus1):
    x = x_ref[...]
    gw = gw_ref[...]
    logits = lax.dot_general(x, gw, (((1,), (1,)), ((), ())),
                             preferred_element_type=jnp.float32)
    col = lax.broadcasted_iota(jnp.int32, logits.shape, 1)
    logits = jnp.where(col >= 1, logits, _NEG)
    m = jnp.max(logits, axis=-1, keepdims=True)
    p = jnp.exp(logits - m)
    p = p / jnp.sum(p, axis=-1, keepdims=True)
    biased = p + bias_ref[...][0:1, :]
    work = biased
    sel = jnp.zeros(biased.shape, dtype=jnp.bool_)
    for _ in range(n_exp_minus1):
        mx = jnp.max(work, axis=-1, keepdims=True)
        is_m = work == mx
        first = jnp.min(jnp.where(is_m, col, 9999), axis=-1, keepdims=True)
        pick = col == first
        sel = jnp.logical_or(sel, pick)
        work = jnp.where(pick, 3.0 * _NEG, work)
    s = jnp.sum(jnp.where(sel, biased, 0.0), axis=-1, keepdims=True)
    frac = n_exp_minus1 / (n_exp_minus1 + 1.0)
    w = jnp.where(sel, biased * (frac / s), 0.0)
    rw_ref[...] = jnp.where(col == 0, 1.0 - frac, w)


def _expert_kernel(rw_ref, x_ref, wfc_ref, wproj_ref, out_ref):
    e = pl.program_id(1)
    x = x_ref[...]
    h = lax.dot_general(x, wfc_ref[0], (((1,), (1,)), ((), ())),
                        preferred_element_type=jnp.float32)
    h = jnp.square(jnp.maximum(h, 0.0)).astype(jnp.bfloat16)
    y = lax.dot_general(h, wproj_ref[0], (((1,), (1,)), ((), ())),
                        preferred_element_type=jnp.float32)
    col = lax.broadcasted_iota(jnp.int32, rw_ref.shape, 1)
    w = jnp.sum(rw_ref[...] * (col == e).astype(jnp.float32), axis=1,
                keepdims=True)
    contrib = y * w

    @pl.when(e == 0)
    def _():
        out_ref[...] = contrib

    @pl.when(e != 0)
    def _():
        out_ref[...] += contrib


def kernel(x, c_fc_w, c_proj_w, gate_w, expert_bias):
    b, t, d = x.shape
    n_exp, h_dim, _ = c_fc_w.shape
    n_tok = b * t
    x_flat = x.reshape(n_tok, d)
    num_exp_m1 = 7

    gw_pad = jnp.concatenate(
        [jnp.zeros((1, d), dtype=gate_w.dtype), gate_w], axis=0)
    bias_pad = jnp.concatenate(
        [jnp.full((1,), _NEG, dtype=expert_bias.dtype), expert_bias])
    bias_pad = jnp.broadcast_to(bias_pad[None, :], (8, n_exp))

    rt = min(1024, n_tok)
    rw = pl.pallas_call(
        functools.partial(_routing_kernel, n_exp_minus1=num_exp_m1),
        grid=(n_tok // rt,),
        in_specs=[
            pl.BlockSpec((rt, d), lambda i: (i, 0)),
            pl.BlockSpec((n_exp, d), lambda i: (0, 0)),
            pl.BlockSpec((8, n_exp), lambda i: (0, 0)),
        ],
        out_specs=pl.BlockSpec((rt, n_exp), lambda i: (i, 0)),
        out_shape=jax.ShapeDtypeStruct((n_tok, n_exp), jnp.float32),
    )(x_flat, gw_pad, bias_pad)

    tt = min(2048, n_tok)
    out = pl.pallas_call(
        _expert_kernel,
        grid=(n_tok // tt, n_exp),
        in_specs=[
            pl.BlockSpec((tt, n_exp), lambda i, e: (i, 0)),
            pl.BlockSpec((tt, d), lambda i, e: (i, 0)),
            pl.BlockSpec((1, h_dim, d), lambda i, e: (e, 0, 0)),
            pl.BlockSpec((1, d, h_dim), lambda i, e: (e, 0, 0)),
        ],
        out_specs=pl.BlockSpec((tt, d), lambda i, e: (i, 0)),
        out_shape=jax.ShapeDtypeStruct((n_tok, d), jnp.float32),
        compiler_params=pltpu.CompilerParams(
            dimension_semantics=("parallel", "arbitrary")),
    )(rw, x_flat.astype(jnp.bfloat16), c_fc_w.astype(jnp.bfloat16),
      c_proj_w.astype(jnp.bfloat16))

    return out.reshape(b, t, d), rw


# speedup vs baseline: 1.1628x; 1.1628x over previous
import functools

import jax
import jax.numpy as jnp
from jax import lax
from jax.experimental import pallas as pl
from jax.experimental.pallas import tpu as pltpu

_NEG = -1e30


def _routing_kernel(x_ref, gw_ref, bias_ref, rw_ref, *, n_exp_minus1):
    x = x_ref[...]
    gw = gw_ref[...]
    logits = lax.dot_general(x, gw, (((1,), (1,)), ((), ())),
                             preferred_element_type=jnp.float32)
    col = lax.broadcasted_iota(jnp.int32, logits.shape, 1)
    logits = jnp.where(col >= 1, logits, _NEG)
    m = jnp.max(logits, axis=-1, keepdims=True)
    p = jnp.exp(logits - m)
    p = p / jnp.sum(p, axis=-1, keepdims=True)
    biased = p + bias_ref[...][0:1, :]
    work = biased
    sel = jnp.zeros(biased.shape, dtype=jnp.bool_)
    for _ in range(n_exp_minus1):
        mx = jnp.max(work, axis=-1, keepdims=True)
        is_m = work == mx
        first = jnp.min(jnp.where(is_m, col, 9999), axis=-1, keepdims=True)
        pick = col == first
        sel = jnp.logical_or(sel, pick)
        work = jnp.where(pick, 3.0 * _NEG, work)
    s = jnp.sum(jnp.where(sel, biased, 0.0), axis=-1, keepdims=True)
    frac = n_exp_minus1 / (n_exp_minus1 + 1.0)
    w = jnp.where(sel, biased * (frac / s), 0.0)
    rw_ref[...] = jnp.where(col == 0, 1.0 - frac, w)


def _expert_kernel(rw_ref, x_ref, wfc_ref, wproj_ref, out_ref):
    e = pl.program_id(1)
    x = x_ref[...]
    h = lax.dot_general(x, wfc_ref[0], (((1,), (1,)), ((), ())),
                        preferred_element_type=jnp.float32)
    h = jnp.square(jnp.maximum(h, 0.0))
    y = lax.dot_general(h, wproj_ref[0], (((1,), (1,)), ((), ())),
                        preferred_element_type=jnp.float32)
    col = lax.broadcasted_iota(jnp.int32, rw_ref.shape, 1)
    w = jnp.sum(rw_ref[...] * (col == e).astype(jnp.float32), axis=1,
                keepdims=True)
    contrib = y * w

    @pl.when(e == 0)
    def _():
        out_ref[...] = contrib

    @pl.when(e != 0)
    def _():
        out_ref[...] += contrib


def kernel(x, c_fc_w, c_proj_w, gate_w, expert_bias):
    b, t, d = x.shape
    n_exp, h_dim, _ = c_fc_w.shape
    n_tok = b * t
    x_flat = x.reshape(n_tok, d)
    num_exp_m1 = 7

    gw_pad = jnp.concatenate(
        [jnp.zeros((1, d), dtype=gate_w.dtype), gate_w], axis=0)
    bias_pad = jnp.concatenate(
        [jnp.full((1,), _NEG, dtype=expert_bias.dtype), expert_bias])
    bias_pad = jnp.broadcast_to(bias_pad[None, :], (8, n_exp))

    rt = min(1024, n_tok)
    rw = pl.pallas_call(
        functools.partial(_routing_kernel, n_exp_minus1=num_exp_m1),
        grid=(n_tok // rt,),
        in_specs=[
            pl.BlockSpec((rt, d), lambda i: (i, 0)),
            pl.BlockSpec((n_exp, d), lambda i: (0, 0)),
            pl.BlockSpec((8, n_exp), lambda i: (0, 0)),
        ],
        out_specs=pl.BlockSpec((rt, n_exp), lambda i: (i, 0)),
        out_shape=jax.ShapeDtypeStruct((n_tok, n_exp), jnp.float32),
    )(x_flat, gw_pad, bias_pad)

    tt = min(4096, n_tok)
    out = pl.pallas_call(
        _expert_kernel,
        grid=(n_tok // tt, n_exp),
        in_specs=[
            pl.BlockSpec((tt, n_exp), lambda i, e: (i, 0)),
            pl.BlockSpec((tt, d), lambda i, e: (i, 0)),
            pl.BlockSpec((1, h_dim, d), lambda i, e: (e, 0, 0)),
            pl.BlockSpec((1, d, h_dim), lambda i, e: (e, 0, 0)),
        ],
        out_specs=pl.BlockSpec((tt, d), lambda i, e: (i, 0)),
        out_shape=jax.ShapeDtypeStruct((n_tok, d), jnp.float32),
        compiler_params=pltpu.CompilerParams(
            dimension_semantics=("parallel", "arbitrary")),
    )(rw, x_flat, c_fc_w, c_proj_w)

    return out.reshape(b, t, d), rw


# speedup vs baseline: 1.1944x; 1.0272x over previous
import functools

import jax
import jax.numpy as jnp
from jax import lax
from jax.experimental import pallas as pl
from jax.experimental.pallas import tpu as pltpu

_NEG = -1e30


def _routing_kernel(x_ref, gw_ref, bias_ref, rw_ref, *, n_exp_minus1):
    x = x_ref[...]
    gw = gw_ref[...]
    logits = lax.dot_general(x, gw, (((1,), (1,)), ((), ())),
                             preferred_element_type=jnp.float32)
    col = lax.broadcasted_iota(jnp.int32, logits.shape, 1)
    logits = jnp.where(col >= 1, logits, _NEG)
    m = jnp.max(logits, axis=-1, keepdims=True)
    p = jnp.exp(logits - m)
    p = p / jnp.sum(p, axis=-1, keepdims=True)
    biased = p + bias_ref[...][0:1, :]
    work = biased
    sel = jnp.zeros(biased.shape, dtype=jnp.bool_)
    for _ in range(n_exp_minus1):
        mx = jnp.max(work, axis=-1, keepdims=True)
        is_m = work == mx
        first = jnp.min(jnp.where(is_m, col, 9999), axis=-1, keepdims=True)
        pick = col == first
        sel = jnp.logical_or(sel, pick)
        work = jnp.where(pick, 3.0 * _NEG, work)
    s = jnp.sum(jnp.where(sel, biased, 0.0), axis=-1, keepdims=True)
    frac = n_exp_minus1 / (n_exp_minus1 + 1.0)
    w = jnp.where(sel, biased * (frac / s), 0.0)
    rw_ref[...] = jnp.where(col == 0, 1.0 - frac, w)


def _expert_kernel(rw_ref, x_ref, wfc_ref, wproj_ref, out_ref):
    e = pl.program_id(1)
    x = x_ref[...]
    h = lax.dot_general(x, wfc_ref[0], (((1,), (1,)), ((), ())),
                        preferred_element_type=jnp.float32)
    h = jnp.square(jnp.maximum(h, 0.0))
    y = lax.dot_general(h, wproj_ref[0], (((1,), (1,)), ((), ())),
                        preferred_element_type=jnp.float32)
    col = lax.broadcasted_iota(jnp.int32, rw_ref.shape, 1)
    w = jnp.sum(rw_ref[...] * (col == e).astype(jnp.float32), axis=1,
                keepdims=True)
    contrib = y * w

    @pl.when(e == 0)
    def _():
        out_ref[...] = contrib

    @pl.when(e != 0)
    def _():
        out_ref[...] += contrib


def kernel(x, c_fc_w, c_proj_w, gate_w, expert_bias):
    b, t, d = x.shape
    n_exp, h_dim, _ = c_fc_w.shape
    n_tok = b * t
    x_flat = x.reshape(n_tok, d)
    num_exp_m1 = 7

    gw_pad = jnp.concatenate(
        [jnp.zeros((1, d), dtype=gate_w.dtype), gate_w], axis=0)
    bias_pad = jnp.concatenate(
        [jnp.full((1,), _NEG, dtype=expert_bias.dtype), expert_bias])
    bias_pad = jnp.broadcast_to(bias_pad[None, :], (8, n_exp))

    rt = min(1024, n_tok)
    rw = pl.pallas_call(
        functools.partial(_routing_kernel, n_exp_minus1=num_exp_m1),
        grid=(n_tok // rt,),
        in_specs=[
            pl.BlockSpec((rt, d), lambda i: (i, 0)),
            pl.BlockSpec((n_exp, d), lambda i: (0, 0)),
            pl.BlockSpec((8, n_exp), lambda i: (0, 0)),
        ],
        out_specs=pl.BlockSpec((rt, n_exp), lambda i: (i, 0)),
        out_shape=jax.ShapeDtypeStruct((n_tok, n_exp), jnp.float32),
    )(x_flat, gw_pad, bias_pad)

    tt = min(8192, n_tok)
    out = pl.pallas_call(
        _expert_kernel,
        grid=(n_tok // tt, n_exp),
        in_specs=[
            pl.BlockSpec((tt, n_exp), lambda i, e: (i, 0)),
            pl.BlockSpec((tt, d), lambda i, e: (i, 0)),
            pl.BlockSpec((1, h_dim, d), lambda i, e: (e, 0, 0)),
            pl.BlockSpec((1, d, h_dim), lambda i, e: (e, 0, 0)),
        ],
        out_specs=pl.BlockSpec((tt, d), lambda i, e: (i, 0)),
        out_shape=jax.ShapeDtypeStruct((n_tok, d), jnp.float32),
        compiler_params=pltpu.CompilerParams(
            dimension_semantics=("parallel", "arbitrary")),
    )(rw, x_flat, c_fc_w, c_proj_w)

    return out.reshape(b, t, d), rw
